# baseline (speedup 1.0000x reference)
import jax
import jax.numpy as jnp
from jax.experimental import pallas as pl

_SEQ_LEN = 50
_LANES = 128
_G = 8
_TPAD = 32


def _body(x_ref, t_ref, o_ref):
    i = pl.program_id(0)
    t = t_ref[...]
    norm = jnp.sqrt(jnp.sum(t * t, axis=-1, keepdims=True))
    scale = jnp.where(norm > 1.0, 1.0 / (norm + 1e-7), 1.0)
    emb = t * scale
    r = jax.lax.broadcasted_iota(jnp.int32, (_G, _TPAD), 0) + i * _G
    k = jax.lax.broadcasted_iota(jnp.int32, (_G, _TPAD), 1)
    oh = (r // _SEQ_LEN == k).astype(jnp.float32)
    sel = jax.lax.dot_general(oh, emb, (((1,), (0,)), ((), ())),
                              preferred_element_type=jnp.float32)
    rep = _LANES // sel.shape[-1]
    e2 = jnp.concatenate([sel] * rep, axis=-1)
    o_ref[...] = x_ref[...] + e2[:, None, :]


def kernel(x, table, num_people):
    del num_people
    R, B, D = x.shape
    F = (B * D) // _LANES
    x3 = x.reshape(R, F, _LANES)
    out = pl.pallas_call(
        _body,
        grid=(R // _G,),
        in_specs=[
            pl.BlockSpec((_G, F, _LANES), lambda i: (i, 0, 0)),
            pl.BlockSpec((_TPAD, D), lambda i: (0, 0)),
        ],
        out_specs=pl.BlockSpec((_G, F, _LANES), lambda i: (i, 0, 0)),
        out_shape=jax.ShapeDtypeStruct((R, F, _LANES), x.dtype),
    )(x3, table)
    return out.reshape(R, B, D)


# speedup vs baseline: 1.0211x; 1.0211x over previous
import jax
import jax.numpy as jnp
from jax.experimental import pallas as pl

_SEQ_LEN = 50
_LANES = 128
_G = 8
_TPAD = 32


def _body(x_ref, t_ref, o_ref):
    i = pl.program_id(0)
    t = t_ref[...]
    norm = jnp.sqrt(jnp.sum(t * t, axis=-1, keepdims=True))
    scale = jnp.where(norm > 1.0, 1.0 / (norm + 1e-7), 1.0)
    emb = t * scale
    r = jax.lax.broadcasted_iota(jnp.int32, (_G, _TPAD), 0) + i * _G
    k = jax.lax.broadcasted_iota(jnp.int32, (_G, _TPAD), 1)
    oh = (r // _SEQ_LEN == k).astype(jnp.float32)
    sel = jax.lax.dot_general(oh, emb, (((1,), (0,)), ((), ())),
                              preferred_element_type=jnp.float32)
    o_ref[...] = x_ref[...] + sel[:, None, :]


def kernel(x, table, num_people):
    del num_people
    R, B, D = x.shape
    return pl.pallas_call(
        _body,
        grid=(R // _G,),
        in_specs=[
            pl.BlockSpec((_G, B, D), lambda i: (i, 0, 0)),
            pl.BlockSpec((_TPAD, D), lambda i: (0, 0)),
        ],
        out_specs=pl.BlockSpec((_G, B, D), lambda i: (i, 0, 0)),
        out_shape=jax.ShapeDtypeStruct((R, B, D), x.dtype),
    )(x, table)


# speedup vs baseline: 1.0216x; 1.0005x over previous
import jax
import jax.numpy as jnp
from jax import lax
from jax.experimental import pallas as pl
from jax.experimental.pallas import tpu as pltpu

_SEQ_LEN = 50
_ROWS = 8
_NBUF = 4
_TPAD = 32


def _body(x_hbm, t_ref, o_hbm, xbuf, obuf, insem, outsem):
    nchunks = x_hbm.shape[0] // _ROWS
    d = t_ref.shape[1]

    t = t_ref[0:_TPAD, :]
    norm = jnp.sqrt(jnp.sum(t * t, axis=-1, keepdims=True))
    scale = jnp.where(norm > 1.0, 1.0 / (norm + 1e-7), 1.0)
    emb = t * scale

    def start_in(c, slot):
        pltpu.make_async_copy(
            x_hbm.at[pl.ds(c * _ROWS, _ROWS)], xbuf.at[slot],
            insem.at[slot]).start()

    for s in range(_NBUF):
        start_in(s, s)

    def step(c, _):
        slot = lax.rem(c, _NBUF)
        pltpu.make_async_copy(
            x_hbm.at[pl.ds(c * _ROWS, _ROWS)], xbuf.at[slot],
            insem.at[slot]).wait()
        r = c * _ROWS + jax.lax.broadcasted_iota(jnp.int32, (_ROWS, _TPAD), 0)
        k = jax.lax.broadcasted_iota(jnp.int32, (_ROWS, _TPAD), 1)
        oh = (r // _SEQ_LEN == k).astype(jnp.float32)
        sel = jax.lax.dot_general(oh, emb, (((1,), (0,)), ((), ())),
                                  preferred_element_type=jnp.float32)

        @pl.when(c >= _NBUF)
        def _():
            pltpu.make_async_copy(
                obuf.at[slot], o_hbm.at[pl.ds((c - _NBUF) * _ROWS, _ROWS)],
                outsem.at[slot]).wait()

        obuf[slot] = xbuf[slot] + sel[:, None, :]
        pltpu.make_async_copy(
            obuf.at[slot], o_hbm.at[pl.ds(c * _ROWS, _ROWS)],
            outsem.at[slot]).start()

        @pl.when(c + _NBUF < nchunks)
        def _():
            start_in(c + _NBUF, slot)
        return 0

    lax.fori_loop(0, nchunks, step, 0)
    for s in range(_NBUF):
        c = nchunks - _NBUF + s
        pltpu.make_async_copy(
            obuf.at[c % _NBUF], o_hbm.at[pl.ds(c * _ROWS, _ROWS)],
            outsem.at[c % _NBUF]).wait()


def kernel(x, table, num_people):
    del num_people
    R, B, D = x.shape
    return pl.pallas_call(
        _body,
        in_specs=[
            pl.BlockSpec(memory_space=pltpu.MemorySpace.HBM),
            pl.BlockSpec(memory_space=pltpu.MemorySpace.VMEM),
        ],
        out_specs=pl.BlockSpec(memory_space=pltpu.MemorySpace.HBM),
        out_shape=jax.ShapeDtypeStruct((R, B, D), x.dtype),
        scratch_shapes=[
            pltpu.VMEM((_NBUF, _ROWS, B, D), jnp.float32),
            pltpu.VMEM((_NBUF, _ROWS, B, D), jnp.float32),
            pltpu.SemaphoreType.DMA((_NBUF,)),
            pltpu.SemaphoreType.DMA((_NBUF,)),
        ],
    )(x, table)


# speedup vs baseline: 5.4638x; 5.3485x over previous
import jax
import jax.numpy as jnp
from jax.experimental import pallas as pl

_SEQ_LEN = 50
_G = 8
_TPAD = 32


def _body(x_ref, t_ref, o_ref):
    c = pl.program_id(0)
    t = t_ref[...]
    norm = jnp.sqrt(jnp.sum(t * t, axis=-1, keepdims=True))
    scale = jnp.where(norm > 1.0, 1.0 / (norm + 1e-7), 1.0)
    emb = t * scale
    r = jax.lax.broadcasted_iota(jnp.int32, (_G, _TPAD), 0) + c * _G
    k = jax.lax.broadcasted_iota(jnp.int32, (_G, _TPAD), 1)
    oh = (r // _SEQ_LEN == k).astype(jnp.float32)
    sel = jax.lax.dot_general(oh, emb, (((1,), (0,)), ((), ())),
                              preferred_element_type=jnp.float32)
    o_ref[...] = x_ref[...] + sel[:, :, None]


def kernel(x, table, num_people):
    del num_people
    R, B, D = x.shape
    xt = jnp.transpose(x, (0, 2, 1))
    out = pl.pallas_call(
        _body,
        grid=(R // _G,),
        in_specs=[
            pl.BlockSpec((_G, D, B), lambda c: (c, 0, 0)),
            pl.BlockSpec((_TPAD, D), lambda c: (0, 0)),
        ],
        out_specs=pl.BlockSpec((_G, D, B), lambda c: (c, 0, 0)),
        out_shape=jax.ShapeDtypeStruct((R, D, B), x.dtype),
    )(xt, table)
    return jnp.transpose(out, (0, 2, 1))


# speedup vs baseline: 6.3299x; 1.1585x over previous
import jax
import jax.numpy as jnp
from jax.experimental import pallas as pl

_SEQ_LEN = 50
_G = 20
_TPAD = 32


def _body(x_ref, t_ref, o_ref):
    c = pl.program_id(0)
    t = t_ref[...]
    norm = jnp.sqrt(jnp.sum(t * t, axis=-1, keepdims=True))
    scale = jnp.where(norm > 1.0, 1.0 / (norm + 1e-7), 1.0)
    emb = t * scale
    r = jax.lax.broadcasted_iota(jnp.int32, (_G, _TPAD), 0) + c * _G
    k = jax.lax.broadcasted_iota(jnp.int32, (_G, _TPAD), 1)
    oh = (r // _SEQ_LEN == k).astype(jnp.float32)
    sel = jax.lax.dot_general(oh, emb, (((1,), (0,)), ((), ())),
                              preferred_element_type=jnp.float32)
    o_ref[...] = x_ref[...] + sel[:, :, None]


def kernel(x, table, num_people):
    del num_people
    R, B, D = x.shape
    xt = jnp.transpose(x, (0, 2, 1))
    out = pl.pallas_call(
        _body,
        grid=(R // _G,),
        in_specs=[
            pl.BlockSpec((_G, D, B), lambda c: (c, 0, 0)),
            pl.BlockSpec((_TPAD, D), lambda c: (0, 0)),
        ],
        out_specs=pl.BlockSpec((_G, D, B), lambda c: (c, 0, 0)),
        out_shape=jax.ShapeDtypeStruct((R, D, B), x.dtype),
    )(xt, table)
    return jnp.transpose(out, (0, 2, 1))


# speedup vs baseline: 6.4136x; 1.0132x over previous
import jax
import jax.numpy as jnp
from jax.experimental import pallas as pl

_SEQ_LEN = 50
_G = 40
_TPAD = 32


def _body(x_ref, t_ref, o_ref):
    c = pl.program_id(0)
    t = t_ref[...]
    norm = jnp.sqrt(jnp.sum(t * t, axis=-1, keepdims=True))
    scale = jnp.where(norm > 1.0, 1.0 / (norm + 1e-7), 1.0)
    emb = t * scale
    r = jax.lax.broadcasted_iota(jnp.int32, (_G, _TPAD), 0) + c * _G
    k = jax.lax.broadcasted_iota(jnp.int32, (_G, _TPAD), 1)
    oh = (r // _SEQ_LEN == k).astype(jnp.float32)
    sel = jax.lax.dot_general(oh, emb, (((1,), (0,)), ((), ())),
                              preferred_element_type=jnp.float32)
    o_ref[...] = x_ref[...] + sel[:, :, None]


def kernel(x, table, num_people):
    del num_people
    R, B, D = x.shape
    xt = jnp.transpose(x, (0, 2, 1))
    out = pl.pallas_call(
        _body,
        grid=(R // _G,),
        in_specs=[
            pl.BlockSpec((_G, D, B), lambda c: (c, 0, 0)),
            pl.BlockSpec((_TPAD, D), lambda c: (0, 0)),
        ],
        out_specs=pl.BlockSpec((_G, D, B), lambda c: (c, 0, 0)),
        out_shape=jax.ShapeDtypeStruct((R, D, B), x.dtype),
    )(xt, table)
    return jnp.transpose(out, (0, 2, 1))


# speedup vs baseline: 6.4204x; 1.0011x over previous
import jax
import jax.numpy as jnp
from jax.experimental import pallas as pl

_SEQ_LEN = 50
_G = 50
_TPAD = 32


def _body(x_ref, t_ref, o_ref):
    c = pl.program_id(0)
    t = t_ref[...]
    norm = jnp.sqrt(jnp.sum(t * t, axis=-1, keepdims=True))
    scale = jnp.where(norm > 1.0, 1.0 / (norm + 1e-7), 1.0)
    emb = t * scale
    r = jax.lax.broadcasted_iota(jnp.int32, (_G, _TPAD), 0) + c * _G
    k = jax.lax.broadcasted_iota(jnp.int32, (_G, _TPAD), 1)
    oh = (r // _SEQ_LEN == k).astype(jnp.float32)
    sel = jax.lax.dot_general(oh, emb, (((1,), (0,)), ((), ())),
                              preferred_element_type=jnp.float32)
    o_ref[...] = x_ref[...] + sel[:, :, None]


def kernel(x, table, num_people):
    del num_people
    R, B, D = x.shape
    xt = jnp.transpose(x, (0, 2, 1))
    out = pl.pallas_call(
        _body,
        grid=(R // _G,),
        in_specs=[
            pl.BlockSpec((_G, D, B), lambda c: (c, 0, 0)),
            pl.BlockSpec((_TPAD, D), lambda c: (0, 0)),
        ],
        out_specs=pl.BlockSpec((_G, D, B), lambda c: (c, 0, 0)),
        out_shape=jax.ShapeDtypeStruct((R, D, B), x.dtype),
    )(xt, table)
    return jnp.transpose(out, (0, 2, 1))


# speedup vs baseline: 6.4998x; 1.0124x over previous
import jax
import jax.numpy as jnp
from jax.experimental import pallas as pl

_SEQ_LEN = 50
_G = 50
_TPAD = 128


def _body(x_ref, t_ref, o_ref):
    c = pl.program_id(0)
    t = t_ref[...]
    norm = jnp.sqrt(jnp.sum(t * t, axis=0, keepdims=True))
    scale = jnp.where(norm > 1.0, 1.0 / (norm + 1e-7), 1.0)
    emb_t = t * scale
    r = jax.lax.broadcasted_iota(jnp.int32, (_G, _TPAD), 0) + c * _G
    k = jax.lax.broadcasted_iota(jnp.int32, (_G, _TPAD), 1)
    oh = (r // _SEQ_LEN == k).astype(jnp.float32)
    sel = jax.lax.dot_general(oh, emb_t, (((1,), (1,)), ((), ())),
                              preferred_element_type=jnp.float32)
    o_ref[...] = x_ref[...] + sel[:, :, None]


def kernel(x, table, num_people):
    del num_people
    R, B, D = x.shape
    xt = jnp.transpose(x, (0, 2, 1))
    tt = jnp.transpose(table, (1, 0))
    out = pl.pallas_call(
        _body,
        grid=(R // _G,),
        in_specs=[
            pl.BlockSpec((_G, D, B), lambda c: (c, 0, 0)),
            pl.BlockSpec((D, _TPAD), lambda c: (0, 0)),
        ],
        out_specs=pl.BlockSpec((_G, D, B), lambda c: (c, 0, 0)),
        out_shape=jax.ShapeDtypeStruct((R, D, B), x.dtype),
    )(xt, tt)
    return jnp.transpose(out, (0, 2, 1))
